# Initial kernel scaffold; baseline (speedup 1.0000x reference)
import jax
import jax.numpy as jnp
from jax.experimental import pallas as pl


def kernel(x, W, b):
    raise NotImplementedError("write your pallas kernel here")


# speedup vs baseline: 1.7366x; 1.7366x over previous
import functools
import math

import jax
import jax.numpy as jnp
import numpy as np
from jax.experimental import pallas as pl

ACTUAL_DIM = 768
PADDED_DIM = 1024
NUM_LEVELS = 16
SIGMA = 1.0
_HALF = (NUM_LEVELS - 1) / 2.0


def _hadamard(n):
    H = np.array([[1.0]], dtype=np.float32)
    while H.shape[0] < n:
        H = np.block([[H, H], [H, -H]]).astype(np.float32)
    return H / np.sqrt(np.float32(n))


_H = _hadamard(PADDED_DIM)
_SIGNS = np.random.RandomState(1234).choice(
    np.array([-1.0, 1.0], dtype=np.float32), size=(PADDED_DIM,)
).astype(np.float32)
_S768 = _SIGNS[:ACTUAL_DIM]
_HS_FWD = (_S768[:, None] * _H[:ACTUAL_DIM, :]).astype(np.float32)
_HS_INV = (_H[:, :ACTUAL_DIM] * _S768[None, :]).astype(np.float32)


def _fused_kernel(x_ref, w_ref, b_ref, hf_ref, hi_ref, out_ref):
    x = x_ref[...]
    y = jnp.dot(x, w_ref[...], preferred_element_type=jnp.float32)
    y = y + b_ref[...]
    r = jnp.dot(y, hf_ref[...], preferred_element_type=jnp.float32)
    q = jnp.clip(jnp.round(r / SIGMA + _HALF), 0.0, NUM_LEVELS - 1.0)
    rq = (q - _HALF) * SIGMA
    dec = jnp.dot(rq, hi_ref[...], preferred_element_type=jnp.float32)
    out_ref[...] = y + (dec - y)


@functools.partial(jax.jit, static_argnames=("block_m",))
def _run(x2d, W, b2d, hf, hi, block_m):
    n_tok = x2d.shape[0]
    grid = (n_tok // block_m,)
    return pl.pallas_call(
        _fused_kernel,
        grid=grid,
        in_specs=[
            pl.BlockSpec((block_m, ACTUAL_DIM), lambda i: (i, 0)),
            pl.BlockSpec((ACTUAL_DIM, ACTUAL_DIM), lambda i: (0, 0)),
            pl.BlockSpec((1, ACTUAL_DIM), lambda i: (0, 0)),
            pl.BlockSpec((ACTUAL_DIM, PADDED_DIM), lambda i: (0, 0)),
            pl.BlockSpec((PADDED_DIM, ACTUAL_DIM), lambda i: (0, 0)),
        ],
        out_specs=pl.BlockSpec((block_m, ACTUAL_DIM), lambda i: (i, 0)),
        out_shape=jax.ShapeDtypeStruct((n_tok, ACTUAL_DIM), jnp.float32),
    )(x2d, W, b2d, hf, hi)


def kernel(x, W, b):
    batch, seq, dim = x.shape
    x2d = x.reshape(batch * seq, dim)
    b2d = b.reshape(1, dim)
    hf = jnp.asarray(_HS_FWD)
    hi = jnp.asarray(_HS_INV)
    out = _run(x2d, W, b2d, hf, hi, 512)
    return out.reshape(batch, seq, dim)


# speedup vs baseline: 2.5431x; 1.4644x over previous
import functools
import math

import jax
import jax.numpy as jnp
import numpy as np
from jax.experimental import pallas as pl

ACTUAL_DIM = 768
PADDED_DIM = 1024
GROUP = 256
NUM_LEVELS = 16
SIGMA = 1.0
_HALF = (NUM_LEVELS - 1) / 2.0


def _hadamard(n):
    H = np.array([[1.0]], dtype=np.float32)
    while H.shape[0] < n:
        H = np.block([[H, H], [H, -H]]).astype(np.float32)
    return H / np.sqrt(np.float32(n))


_H = _hadamard(PADDED_DIM)
_SIGNS = np.random.RandomState(1234).choice(
    np.array([-1.0, 1.0], dtype=np.float32), size=(PADDED_DIM,)
).astype(np.float32)

_H256 = (_hadamard(GROUP) * (np.sqrt(np.float32(GROUP)) / 32.0)).astype(np.float32)

_HF = np.stack([
    _SIGNS[a * GROUP:(a + 1) * GROUP][:, None] * _H256 for a in range(3)
], axis=0).astype(np.float32)

_SOUT = _SIGNS[:ACTUAL_DIM].reshape(3, 1, GROUP).astype(np.float32)


def _fused_kernel(x_ref, w_ref, b_ref, hf_ref, hm_ref, s_ref, out_ref):
    y = jnp.dot(x_ref[...], w_ref[...], preferred_element_type=jnp.float32)
    y = y + b_ref[...]

    p = [
        jnp.dot(y[:, a * GROUP:(a + 1) * GROUP], hf_ref[a],
                preferred_element_type=jnp.float32)
        for a in range(3)
    ]
    a0 = p[0] + p[1]
    a1 = p[0] - p[1]
    r = [a0 + p[2], a1 + p[2], a0 - p[2], a1 - p[2]]

    rq = [
        (jnp.clip(jnp.round(rg / SIGMA + _HALF), 0.0, NUM_LEVELS - 1.0) - _HALF)
        * SIGMA
        for rg in r
    ]

    q = [
        jnp.dot(rqg, hm_ref[...], preferred_element_type=jnp.float32)
        for rqg in rq
    ]
    b0 = q[0] + q[1]
    b1 = q[0] - q[1]
    b2 = q[2] + q[3]
    b3 = q[2] - q[3]
    d0 = (b0 + b2) * s_ref[0]
    d1 = (b1 + b3) * s_ref[1]
    d2 = (b0 - b2) * s_ref[2]
    dec = jnp.concatenate([d0, d1, d2], axis=1)
    out_ref[...] = y + (dec - y)


@functools.partial(jax.jit, static_argnames=("block_m",))
def _run(x2d, W, b2d, hf, hm, souts, block_m):
    n_tok = x2d.shape[0]
    grid = (n_tok // block_m,)
    return pl.pallas_call(
        _fused_kernel,
        grid=grid,
        in_specs=[
            pl.BlockSpec((block_m, ACTUAL_DIM), lambda i: (i, 0)),
            pl.BlockSpec((ACTUAL_DIM, ACTUAL_DIM), lambda i: (0, 0)),
            pl.BlockSpec((1, ACTUAL_DIM), lambda i: (0, 0)),
            pl.BlockSpec((3, GROUP, GROUP), lambda i: (0, 0, 0)),
            pl.BlockSpec((GROUP, GROUP), lambda i: (0, 0)),
            pl.BlockSpec((3, 1, GROUP), lambda i: (0, 0, 0)),
        ],
        out_specs=pl.BlockSpec((block_m, ACTUAL_DIM), lambda i: (i, 0)),
        out_shape=jax.ShapeDtypeStruct((n_tok, ACTUAL_DIM), jnp.float32),
    )(x2d, W, b2d, hf, hm, souts)


def kernel(x, W, b):
    batch, seq, dim = x.shape
    x2d = x.reshape(batch * seq, dim)
    b2d = b.reshape(1, dim)
    hf = jnp.asarray(_HF)
    hm = jnp.asarray(_H256)
    souts = jnp.asarray(_SOUT)
    out = _run(x2d, W, b2d, hf, hm, souts, 512)
    return out.reshape(batch, seq, dim)


# speedup vs baseline: 2.9569x; 1.1627x over previous
import functools
import math

import jax
import jax.numpy as jnp
import numpy as np
from jax.experimental import pallas as pl

ACTUAL_DIM = 768
PADDED_DIM = 1024
GROUP = 256
NUM_LEVELS = 16
SIGMA = 1.0
_HALF = (NUM_LEVELS - 1) / 2.0


def _hadamard(n):
    H = np.array([[1.0]], dtype=np.float32)
    while H.shape[0] < n:
        H = np.block([[H, H], [H, -H]]).astype(np.float32)
    return H / np.sqrt(np.float32(n))


_H = _hadamard(PADDED_DIM)
_SIGNS = np.random.RandomState(1234).choice(
    np.array([-1.0, 1.0], dtype=np.float32), size=(PADDED_DIM,)
).astype(np.float32)

_H256 = (_hadamard(GROUP) * (np.sqrt(np.float32(GROUP)) / 32.0)).astype(np.float32)

_HF = np.stack([
    _SIGNS[a * GROUP:(a + 1) * GROUP][:, None] * _H256 for a in range(3)
], axis=0).astype(np.float32)

_SOUT = _SIGNS[:ACTUAL_DIM].reshape(3, 1, GROUP).astype(np.float32)


def _fused_kernel(x_ref, w_ref, b_ref, hf_ref, hm_ref, s_ref, out_ref):
    y = jnp.dot(x_ref[...], w_ref[...], preferred_element_type=jnp.float32)
    y = y + b_ref[...]

    p = [
        jnp.dot(y[:, a * GROUP:(a + 1) * GROUP], hf_ref[a],
                preferred_element_type=jnp.float32)
        for a in range(3)
    ]
    a0 = p[0] + p[1]
    a1 = p[0] - p[1]
    r = [a0 + p[2], a1 + p[2], a0 - p[2], a1 - p[2]]

    rq = [
        (jnp.clip(jnp.round(rg / SIGMA + _HALF), 0.0, NUM_LEVELS - 1.0) - _HALF)
        * SIGMA
        for rg in r
    ]

    q = [
        jnp.dot(rqg, hm_ref[...], preferred_element_type=jnp.float32)
        for rqg in rq
    ]
    b0 = q[0] + q[1]
    b1 = q[0] - q[1]
    b2 = q[2] + q[3]
    b3 = q[2] - q[3]
    d0 = (b0 + b2) * s_ref[0]
    d1 = (b1 + b3) * s_ref[1]
    d2 = (b0 - b2) * s_ref[2]
    dec = jnp.concatenate([d0, d1, d2], axis=1)
    out_ref[...] = y + (dec - y)


@functools.partial(jax.jit, static_argnames=("block_m",))
def _run(x2d, W, b2d, hf, hm, souts, block_m):
    n_tok = x2d.shape[0]
    grid = (n_tok // block_m,)
    return pl.pallas_call(
        _fused_kernel,
        grid=grid,
        in_specs=[
            pl.BlockSpec((block_m, ACTUAL_DIM), lambda i: (i, 0)),
            pl.BlockSpec((ACTUAL_DIM, ACTUAL_DIM), lambda i: (0, 0)),
            pl.BlockSpec((1, ACTUAL_DIM), lambda i: (0, 0)),
            pl.BlockSpec((3, GROUP, GROUP), lambda i: (0, 0, 0)),
            pl.BlockSpec((GROUP, GROUP), lambda i: (0, 0)),
            pl.BlockSpec((3, 1, GROUP), lambda i: (0, 0, 0)),
        ],
        out_specs=pl.BlockSpec((block_m, ACTUAL_DIM), lambda i: (i, 0)),
        out_shape=jax.ShapeDtypeStruct((n_tok, ACTUAL_DIM), jnp.float32),
    )(x2d, W, b2d, hf, hm, souts)


def kernel(x, W, b):
    batch, seq, dim = x.shape
    x2d = x.reshape(batch * seq, dim)
    b2d = b.reshape(1, dim)
    hf = jnp.asarray(_HF)
    hm = jnp.asarray(_H256)
    souts = jnp.asarray(_SOUT)
    out = _run(x2d, W, b2d, hf, hm, souts, 1024)
    return out.reshape(batch, seq, dim)


# speedup vs baseline: 2.9945x; 1.0127x over previous
import functools
import math

import jax
import jax.numpy as jnp
import numpy as np
from jax.experimental import pallas as pl

ACTUAL_DIM = 768
PADDED_DIM = 1024
GROUP = 256
NUM_LEVELS = 16
SIGMA = 1.0
_HALF = (NUM_LEVELS - 1) / 2.0


def _hadamard(n):
    H = np.array([[1.0]], dtype=np.float32)
    while H.shape[0] < n:
        H = np.block([[H, H], [H, -H]]).astype(np.float32)
    return H / np.sqrt(np.float32(n))


_H = _hadamard(PADDED_DIM)
_SIGNS = np.random.RandomState(1234).choice(
    np.array([-1.0, 1.0], dtype=np.float32), size=(PADDED_DIM,)
).astype(np.float32)

_H256 = (_hadamard(GROUP) * (np.sqrt(np.float32(GROUP)) / 32.0)).astype(np.float32)

_HF = np.stack([
    _SIGNS[a * GROUP:(a + 1) * GROUP][:, None] * _H256 for a in range(3)
], axis=0).astype(np.float32)

_SOUT = _SIGNS[:ACTUAL_DIM].reshape(3, 1, GROUP).astype(np.float32)


def _fused_kernel(x_ref, w_ref, b_ref, hf_ref, hm_ref, s_ref, out_ref):
    y = jnp.dot(x_ref[...], w_ref[...], preferred_element_type=jnp.float32)
    y = y + b_ref[...]

    p = [
        jnp.dot(y[:, a * GROUP:(a + 1) * GROUP], hf_ref[a],
                preferred_element_type=jnp.float32)
        for a in range(3)
    ]
    a0 = p[0] + p[1]
    a1 = p[0] - p[1]
    r = [a0 + p[2], a1 + p[2], a0 - p[2], a1 - p[2]]

    rq = [
        (jnp.clip(jnp.round(rg / SIGMA + _HALF), 0.0, NUM_LEVELS - 1.0) - _HALF)
        * SIGMA
        for rg in r
    ]

    q = [
        jnp.dot(rqg, hm_ref[...], preferred_element_type=jnp.float32)
        for rqg in rq
    ]
    b0 = q[0] + q[1]
    b1 = q[0] - q[1]
    b2 = q[2] + q[3]
    b3 = q[2] - q[3]
    d0 = (b0 + b2) * s_ref[0]
    d1 = (b1 + b3) * s_ref[1]
    d2 = (b0 - b2) * s_ref[2]
    dec = jnp.concatenate([d0, d1, d2], axis=1)
    out_ref[...] = y + (dec - y)


@functools.partial(jax.jit, static_argnames=("block_m",))
def _run(x2d, W, b2d, hf, hm, souts, block_m):
    n_tok = x2d.shape[0]
    grid = (n_tok // block_m,)
    return pl.pallas_call(
        _fused_kernel,
        grid=grid,
        in_specs=[
            pl.BlockSpec((block_m, ACTUAL_DIM), lambda i: (i, 0)),
            pl.BlockSpec((ACTUAL_DIM, ACTUAL_DIM), lambda i: (0, 0)),
            pl.BlockSpec((1, ACTUAL_DIM), lambda i: (0, 0)),
            pl.BlockSpec((3, GROUP, GROUP), lambda i: (0, 0, 0)),
            pl.BlockSpec((GROUP, GROUP), lambda i: (0, 0)),
            pl.BlockSpec((3, 1, GROUP), lambda i: (0, 0, 0)),
        ],
        out_specs=pl.BlockSpec((block_m, ACTUAL_DIM), lambda i: (i, 0)),
        out_shape=jax.ShapeDtypeStruct((n_tok, ACTUAL_DIM), jnp.float32),
    )(x2d, W, b2d, hf, hm, souts)


def kernel(x, W, b):
    batch, seq, dim = x.shape
    x2d = x.reshape(batch * seq, dim)
    b2d = b.reshape(1, dim)
    hf = jnp.asarray(_HF)
    hm = jnp.asarray(_H256)
    souts = jnp.asarray(_SOUT)
    out = _run(x2d, W, b2d, hf, hm, souts, 2048)
    return out.reshape(batch, seq, dim)
